# baseline (speedup 1.0000x reference)
import jax
import jax.numpy as jnp
import numpy as np
from jax import lax
from jax.experimental import pallas as pl
from jax.experimental.pallas import tpu as pltpu
from jax.experimental.pallas import tpu_sc as plsc

N = 10000
E = 320000
F = 128
H = 128
OUTD = 2
G = 64

NC = 2
NS = 16
NW = NC * NS

K = 128
CPT = 80
EPAD = NW * CPT * K
PADR = 240
AR = N + PADR
RPT = AR // NS

_mesh = plsc.VectorSubcoreMesh(core_axis_name="c", subcore_axis_name="s")


def _sc_deg_body(edges, zdeg, onesh, out, dst_v, ones_v, acc, sem):
    c = lax.axis_index("c")
    s = lax.axis_index("s")
    wid = c * NS + s
    pltpu.sync_copy(edges.at[1, pl.ds(wid * CPT, CPT)], dst_v)
    pltpu.sync_copy(onesh, ones_v)
    pltpu.sync_copy(zdeg.at[pl.ds(s * RPT, RPT)], acc.at[pl.ds(s * RPT, RPT)])
    plsc.subcore_barrier()

    def body(j, carry):
        pltpu.async_copy(ones_v, acc.at[dst_v.at[j]], sem, add=True)

        @pl.when(j >= 7)
        def _drain():
            pltpu.make_async_copy(ones_v, acc.at[dst_v.at[j - 7]], sem).wait()

        return carry

    lax.fori_loop(0, CPT, body, 0)
    for t in range(7):
        pltpu.make_async_copy(ones_v, acc.at[dst_v.at[CPT - 7 + t]],
                              sem).wait()
    plsc.subcore_barrier()
    pltpu.sync_copy(acc.at[pl.ds(s * RPT, RPT)],
                    out.at[pl.ds(c * AR + s * RPT, RPT)])


_sc_deg = pl.kernel(
    _sc_deg_body,
    out_type=jax.ShapeDtypeStruct((NC * AR,), jnp.float32),
    mesh=_mesh,
    scratch_types=[
        pltpu.VMEM((CPT, K), jnp.int32),
        pltpu.VMEM((K,), jnp.float32),
        pltpu.VMEM_SHARED((AR,), jnp.float32),
        pltpu.SemaphoreType.DMA,
    ],
)


def _sc_scatter_body(hs, edges, zrows, out, src_v, dst_v, src_w, dst_w,
                     rows0, rows1, acc, sem0, sem1, isem):
    c = lax.axis_index("c")
    s = lax.axis_index("s")
    wid = c * NS + s

    rows = (rows0, rows1)
    sems = (sem0, sem1)
    srcs = (src_v, src_w)
    dsts = (dst_v, dst_w)
    QN = 5
    QC = CPT // QN
    base0 = wid * CPT
    pltpu.sync_copy(edges.at[0, pl.ds(base0, QC)], src_v)
    pltpu.sync_copy(edges.at[1, pl.ds(base0, QC)], dst_v)
    pltpu.async_copy(hs.at[src_v.at[0]], rows0, sem0)
    @pl.when(jnp.logical_and(c == 0, s < NS - 1))
    def _preload():
        pltpu.sync_copy(hs.at[pl.ds(s * RPT, RPT)],
                        acc.at[pl.ds(s * RPT, RPT)])

    @pl.when(jnp.logical_and(c == 0, s == NS - 1))
    def _preload_tail():
        pltpu.sync_copy(hs.at[pl.ds((NS - 1) * RPT, N - (NS - 1) * RPT)],
                        acc.at[pl.ds((NS - 1) * RPT, N - (NS - 1) * RPT)])
        pltpu.sync_copy(zrows.at[pl.ds(0, AR - N)],
                        acc.at[pl.ds(N, AR - N)])

    @pl.when(c == 1)
    def _zero():
        pltpu.sync_copy(zrows, acc.at[pl.ds(s * RPT, RPT)])

    plsc.subcore_barrier()

    for q in range(QN):
        qb = q & 1
        sq, dq = srcs[qb], dsts[qb]
        if q < QN - 1:
            nbase = wid * CPT + (q + 1) * QC
            pltpu.async_copy(edges.at[0, pl.ds(nbase, QC)], srcs[1 - qb],
                             isem)
            pltpu.async_copy(edges.at[1, pl.ds(nbase, QC)], dsts[1 - qb],
                             isem)

        def body(g, carry):
            for b in range(2):
                j = 2 * g + b
                pltpu.make_async_copy(hs.at[sq.at[j]], rows[b],
                                      sems[b]).wait()

                @pl.when(j + 1 < QC)
                def _prefetch():
                    pltpu.async_copy(hs.at[sq.at[j + 1]], rows[1 - b],
                                     sems[1 - b])

                if q < QN - 1:
                    @pl.when(j == QC - 1)
                    def _prefetch_next_quarter():
                        pltpu.make_async_copy(
                            edges.at[0, pl.ds(nbase, QC)], srcs[1 - qb],
                            isem).wait()
                        pltpu.make_async_copy(
                            edges.at[1, pl.ds(nbase, QC)], dsts[1 - qb],
                            isem).wait()
                        pltpu.async_copy(hs.at[srcs[1 - qb].at[0]],
                                         rows[1 - b], sems[1 - b])

                pltpu.sync_copy(rows[b], acc.at[dq.at[j]], add=True)
            return carry

        lax.fori_loop(0, QC // 2, body, 0)
    plsc.subcore_barrier()
    pltpu.sync_copy(acc.at[pl.ds(s * RPT, RPT)], out.at[c, pl.ds(s * RPT, RPT)])


_sc_scatter = pl.kernel(
    _sc_scatter_body,
    out_type=jax.ShapeDtypeStruct((NC, AR, H), jnp.float32),
    mesh=_mesh,
    scratch_types=[
        pltpu.VMEM((CPT // 5, K), jnp.int32),
        pltpu.VMEM((CPT // 5, K), jnp.int32),
        pltpu.VMEM((CPT // 5, K), jnp.int32),
        pltpu.VMEM((CPT // 5, K), jnp.int32),
        pltpu.VMEM((K, H), jnp.float32),
        pltpu.VMEM((K, H), jnp.float32),
        pltpu.VMEM_SHARED((AR, H), jnp.float32),
        pltpu.SemaphoreType.DMA,
        pltpu.SemaphoreType.DMA,
        pltpu.SemaphoreType.DMA,
    ],
)


BLK = 2000


def _k1_body(x_ref, w_ref, deg_ref, y_ref):
    h = jnp.dot(x_ref[...], w_ref[...], preferred_element_type=jnp.float32)
    dis = lax.rsqrt(deg_ref[...])
    y_ref[...] = h * dis


def _k1(x, W1, deg2d):
    return pl.pallas_call(
        _k1_body,
        grid=(N // BLK,),
        in_specs=[
            pl.BlockSpec((BLK, F), lambda i: (i, 0)),
            pl.BlockSpec((F, H), lambda i: (0, 0)),
            pl.BlockSpec((BLK, 1), lambda i: (i, 0)),
        ],
        out_specs=pl.BlockSpec((BLK, H), lambda i: (i, 0)),
        out_shape=jax.ShapeDtypeStruct((N, H), jnp.float32),
    )(x, W1, deg2d)


def _k2_body(a_ref, deg_ref, b1_ref, w2_ref, y2_ref):
    dis = lax.rsqrt(deg_ref[...])
    agg = a_ref[0] + a_ref[1]
    x2 = jnp.maximum(agg * dis + b1_ref[...], 0.0)
    y2_ref[...] = jnp.dot(x2, w2_ref[...], preferred_element_type=jnp.float32) * dis


def _k2(agg, deg2d, b1r, W2):
    return pl.pallas_call(
        _k2_body,
        grid=(N // BLK,),
        in_specs=[
            pl.BlockSpec((NC, BLK, H), lambda i: (0, i, 0)),
            pl.BlockSpec((BLK, 1), lambda i: (i, 0)),
            pl.BlockSpec((1, H), lambda i: (0, 0)),
            pl.BlockSpec((H, H), lambda i: (0, 0)),
        ],
        out_specs=pl.BlockSpec((BLK, H), lambda i: (i, 0)),
        out_shape=jax.ShapeDtypeStruct((N, H), jnp.float32),
    )(agg, deg2d, b1r, W2)


def _k3_body(a_ref, deg_ref, b2_ref, batch_ref, wlin_ref, blin_ref,
             out_ref, acc_ref, cnt_ref):
    i = pl.program_id(0)

    @pl.when(i == 0)
    def _init():
        acc_ref[...] = jnp.zeros_like(acc_ref)
        cnt_ref[...] = jnp.zeros_like(cnt_ref)

    dis = lax.rsqrt(deg_ref[...])
    x3 = jnp.maximum((a_ref[0] + a_ref[1]) * dis + b2_ref[...], 0.0)
    bid = batch_ref[...]
    gi = lax.broadcasted_iota(jnp.int32, (BLK, G), 1)
    oh = (bid == gi).astype(jnp.float32)
    acc_ref[...] += lax.dot_general(
        oh, x3, (((0,), (0,)), ((), ())), preferred_element_type=jnp.float32)
    cnt_ref[...] += lax.dot_general(
        oh, jnp.ones((BLK, H), jnp.float32), (((0,), (0,)), ((), ())),
        preferred_element_type=jnp.float32)

    @pl.when(i == pl.num_programs(0) - 1)
    def _fin():
        pooled = acc_ref[...] / jnp.maximum(cnt_ref[...], 1.0)
        out_ref[...] = jnp.dot(pooled, wlin_ref[...],
                               preferred_element_type=jnp.float32) + blin_ref[...]


def _k3(agg, deg2d, b2r, batch2d, wlin_p, blin_p):
    return pl.pallas_call(
        _k3_body,
        grid=(N // BLK,),
        in_specs=[
            pl.BlockSpec((NC, BLK, H), lambda i: (0, i, 0)),
            pl.BlockSpec((BLK, 1), lambda i: (i, 0)),
            pl.BlockSpec((1, H), lambda i: (0, 0)),
            pl.BlockSpec((BLK, 1), lambda i: (i, 0)),
            pl.BlockSpec((H, H), lambda i: (0, 0)),
            pl.BlockSpec((1, H), lambda i: (0, 0)),
        ],
        out_specs=pl.BlockSpec((G, H), lambda i: (0, 0)),
        out_shape=jax.ShapeDtypeStruct((G, H), jnp.float32),
        scratch_shapes=[
            pltpu.VMEM((G, H), jnp.float32),
            pltpu.VMEM((G, H), jnp.float32),
        ],
    )(agg, deg2d, b2r, batch2d, wlin_p, blin_p)


def kernel(x, edge_index, batch, W1, b1, W2, b2, Wlin, blin):
    pi = np.arange(EPAD - E)
    padc = np.stack([pi % K, N + pi % PADR]).astype(np.int32)
    edges3d = jnp.concatenate([edge_index.astype(jnp.int32), padc],
                              axis=1).reshape(2, NW * CPT, K)
    zdeg = jnp.zeros((AR,), jnp.float32)
    zrows = jnp.zeros((RPT, H), jnp.float32)
    onesh = jnp.ones((K,), jnp.float32)

    degp = _sc_deg(edges3d, zdeg, onesh).reshape(NC, AR)
    deg2d = (degp[0, :N] + degp[1, :N] + 1.0).reshape(N, 1)

    y1 = _k1(x, W1, deg2d)
    agg1 = _sc_scatter(y1, edges3d, zrows)

    b1r = b1.reshape(1, H)
    y2 = _k2(agg1, deg2d, b1r, W2)
    agg2 = _sc_scatter(y2, edges3d, zrows)

    b2r = b2.reshape(1, H)
    batch2d = batch.astype(jnp.int32).reshape(N, 1)
    wlin_p = jnp.zeros((H, H), jnp.float32).at[:, :OUTD].set(Wlin)
    blin_p = jnp.zeros((1, H), jnp.float32).at[0, :OUTD].set(blin)
    outp = _k3(agg2, deg2d, b2r, batch2d, wlin_p, blin_p)
    return outp[:, :OUTD]


# speedup vs baseline: 1.0072x; 1.0072x over previous
import jax
import jax.numpy as jnp
import numpy as np
from jax import lax
from jax.experimental import pallas as pl
from jax.experimental.pallas import tpu as pltpu
from jax.experimental.pallas import tpu_sc as plsc

N = 10000
E = 320000
F = 128
H = 128
OUTD = 2
G = 64

NC = 2
NS = 16
NW = NC * NS

K = 128
CPT = 80
EPAD = NW * CPT * K
PADR = 240
AR = N + PADR
RPT = AR // NS

_mesh = plsc.VectorSubcoreMesh(core_axis_name="c", subcore_axis_name="s")


def _sc_deg_body(edges, zdeg, onesh, out, dst_v, ones_v, acc, sem):
    c = lax.axis_index("c")
    s = lax.axis_index("s")
    wid = c * NS + s
    pltpu.sync_copy(edges.at[1, pl.ds(wid * CPT, CPT)], dst_v)
    pltpu.sync_copy(onesh, ones_v)
    pltpu.sync_copy(zdeg.at[pl.ds(s * RPT, RPT)], acc.at[pl.ds(s * RPT, RPT)])
    plsc.subcore_barrier()

    def body(j, carry):
        pltpu.async_copy(ones_v, acc.at[dst_v.at[j]], sem, add=True)

        @pl.when(j >= 7)
        def _drain():
            pltpu.make_async_copy(ones_v, acc.at[dst_v.at[j - 7]], sem).wait()

        return carry

    lax.fori_loop(0, CPT, body, 0)
    for t in range(7):
        pltpu.make_async_copy(ones_v, acc.at[dst_v.at[CPT - 7 + t]],
                              sem).wait()
    plsc.subcore_barrier()
    pltpu.sync_copy(acc.at[pl.ds(s * RPT, RPT)],
                    out.at[pl.ds(c * AR + s * RPT, RPT)])


_sc_deg = pl.kernel(
    _sc_deg_body,
    out_type=jax.ShapeDtypeStruct((NC * AR,), jnp.float32),
    mesh=_mesh,
    scratch_types=[
        pltpu.VMEM((CPT, K), jnp.int32),
        pltpu.VMEM((K,), jnp.float32),
        pltpu.VMEM_SHARED((AR,), jnp.float32),
        pltpu.SemaphoreType.DMA,
    ],
)


def _sc_scatter_body(hs, edges, zrows, out, src_v, dst_v, rows0, rows1,
                     acc, sem0, sem1):
    c = lax.axis_index("c")
    s = lax.axis_index("s")
    wid = c * NS + s

    rows = (rows0, rows1)
    sems = (sem0, sem1)
    HC = CPT // 2
    for half in range(2):
        base = wid * CPT + half * HC
        pltpu.sync_copy(edges.at[0, pl.ds(base, HC)], src_v)
        pltpu.sync_copy(edges.at[1, pl.ds(base, HC)], dst_v)
        pltpu.async_copy(hs.at[src_v.at[0]], rows0, sem0)
        if half == 0:
            @pl.when(jnp.logical_and(c == 0, s < NS - 1))
            def _preload():
                pltpu.sync_copy(hs.at[pl.ds(s * RPT, RPT)],
                                acc.at[pl.ds(s * RPT, RPT)])

            @pl.when(jnp.logical_and(c == 0, s == NS - 1))
            def _preload_tail():
                pltpu.sync_copy(hs.at[pl.ds((NS - 1) * RPT, N - (NS - 1) * RPT)],
                                acc.at[pl.ds((NS - 1) * RPT, N - (NS - 1) * RPT)])
                pltpu.sync_copy(zrows.at[pl.ds(0, AR - N)],
                                acc.at[pl.ds(N, AR - N)])

            @pl.when(c == 1)
            def _zero():
                pltpu.sync_copy(zrows, acc.at[pl.ds(s * RPT, RPT)])

            plsc.subcore_barrier()

        def body(g, carry):
            for b in range(2):
                j = 2 * g + b
                pltpu.make_async_copy(hs.at[src_v.at[j]], rows[b],
                                      sems[b]).wait()

                @pl.when(j + 1 < HC)
                def _prefetch():
                    pltpu.async_copy(hs.at[src_v.at[j + 1]], rows[1 - b],
                                     sems[1 - b])

                pltpu.sync_copy(rows[b], acc.at[dst_v.at[j]], add=True)
            return carry

        lax.fori_loop(0, HC // 2, body, 0)
    plsc.subcore_barrier()
    pltpu.sync_copy(acc.at[pl.ds(s * RPT, RPT)], out.at[c, pl.ds(s * RPT, RPT)])


_sc_scatter = pl.kernel(
    _sc_scatter_body,
    out_type=jax.ShapeDtypeStruct((NC, AR, H), jnp.float32),
    mesh=_mesh,
    scratch_types=[
        pltpu.VMEM((CPT // 2, K), jnp.int32),
        pltpu.VMEM((CPT // 2, K), jnp.int32),
        pltpu.VMEM((K, H), jnp.float32),
        pltpu.VMEM((K, H), jnp.float32),
        pltpu.VMEM_SHARED((AR, H), jnp.float32),
        pltpu.SemaphoreType.DMA,
        pltpu.SemaphoreType.DMA,
    ],
)


BLK = 2000


def _k1_body(x_ref, w_ref, deg_ref, y_ref):
    h = jnp.dot(x_ref[...], w_ref[...], preferred_element_type=jnp.float32)
    dis = lax.rsqrt(deg_ref[...])
    y_ref[...] = h * dis


def _k1(x, W1, deg2d):
    return pl.pallas_call(
        _k1_body,
        grid=(N // BLK,),
        in_specs=[
            pl.BlockSpec((BLK, F), lambda i: (i, 0)),
            pl.BlockSpec((F, H), lambda i: (0, 0)),
            pl.BlockSpec((BLK, 1), lambda i: (i, 0)),
        ],
        out_specs=pl.BlockSpec((BLK, H), lambda i: (i, 0)),
        out_shape=jax.ShapeDtypeStruct((N, H), jnp.float32),
    )(x, W1, deg2d)


def _k2_body(a_ref, deg_ref, b1_ref, w2_ref, y2_ref):
    dis = lax.rsqrt(deg_ref[...])
    agg = a_ref[0] + a_ref[1]
    x2 = jnp.maximum(agg * dis + b1_ref[...], 0.0)
    y2_ref[...] = jnp.dot(x2, w2_ref[...], preferred_element_type=jnp.float32) * dis


def _k2(agg, deg2d, b1r, W2):
    return pl.pallas_call(
        _k2_body,
        grid=(N // BLK,),
        in_specs=[
            pl.BlockSpec((NC, BLK, H), lambda i: (0, i, 0)),
            pl.BlockSpec((BLK, 1), lambda i: (i, 0)),
            pl.BlockSpec((1, H), lambda i: (0, 0)),
            pl.BlockSpec((H, H), lambda i: (0, 0)),
        ],
        out_specs=pl.BlockSpec((BLK, H), lambda i: (i, 0)),
        out_shape=jax.ShapeDtypeStruct((N, H), jnp.float32),
    )(agg, deg2d, b1r, W2)


def _k3_body(a_ref, deg_ref, b2_ref, batch_ref, wlin_ref, blin_ref,
             out_ref, acc_ref, cnt_ref):
    i = pl.program_id(0)

    @pl.when(i == 0)
    def _init():
        acc_ref[...] = jnp.zeros_like(acc_ref)
        cnt_ref[...] = jnp.zeros_like(cnt_ref)

    dis = lax.rsqrt(deg_ref[...])
    x3 = jnp.maximum((a_ref[0] + a_ref[1]) * dis + b2_ref[...], 0.0)
    bid = batch_ref[...]
    gi = lax.broadcasted_iota(jnp.int32, (BLK, G), 1)
    oh = (bid == gi).astype(jnp.float32)
    acc_ref[...] += lax.dot_general(
        oh, x3, (((0,), (0,)), ((), ())), preferred_element_type=jnp.float32)
    cnt_ref[...] += lax.dot_general(
        oh, jnp.ones((BLK, H), jnp.float32), (((0,), (0,)), ((), ())),
        preferred_element_type=jnp.float32)

    @pl.when(i == pl.num_programs(0) - 1)
    def _fin():
        pooled = acc_ref[...] / jnp.maximum(cnt_ref[...], 1.0)
        out_ref[...] = jnp.dot(pooled, wlin_ref[...],
                               preferred_element_type=jnp.float32) + blin_ref[...]


def _k3(agg, deg2d, b2r, batch2d, wlin_p, blin_p):
    return pl.pallas_call(
        _k3_body,
        grid=(N // BLK,),
        in_specs=[
            pl.BlockSpec((NC, BLK, H), lambda i: (0, i, 0)),
            pl.BlockSpec((BLK, 1), lambda i: (i, 0)),
            pl.BlockSpec((1, H), lambda i: (0, 0)),
            pl.BlockSpec((BLK, 1), lambda i: (i, 0)),
            pl.BlockSpec((H, H), lambda i: (0, 0)),
            pl.BlockSpec((1, H), lambda i: (0, 0)),
        ],
        out_specs=pl.BlockSpec((G, H), lambda i: (0, 0)),
        out_shape=jax.ShapeDtypeStruct((G, H), jnp.float32),
        scratch_shapes=[
            pltpu.VMEM((G, H), jnp.float32),
            pltpu.VMEM((G, H), jnp.float32),
        ],
    )(agg, deg2d, b2r, batch2d, wlin_p, blin_p)


def kernel(x, edge_index, batch, W1, b1, W2, b2, Wlin, blin):
    pi = np.arange(EPAD - E)
    padc = np.stack([pi % K, N + pi % PADR]).astype(np.int32)
    edges3d = jnp.concatenate([edge_index.astype(jnp.int32), padc],
                              axis=1).reshape(2, NW * CPT, K)
    zdeg = jnp.zeros((AR,), jnp.float32)
    zrows = jnp.zeros((RPT, H), jnp.float32)
    onesh = jnp.ones((K,), jnp.float32)

    degp = _sc_deg(edges3d, zdeg, onesh).reshape(NC, AR)
    deg2d = (degp[0, :N] + degp[1, :N] + 1.0).reshape(N, 1)

    y1 = _k1(x, W1, deg2d)
    agg1 = _sc_scatter(y1, edges3d, zrows)

    b1r = b1.reshape(1, H)
    y2 = _k2(agg1, deg2d, b1r, W2)
    agg2 = _sc_scatter(y2, edges3d, zrows)

    b2r = b2.reshape(1, H)
    batch2d = batch.astype(jnp.int32).reshape(N, 1)
    wlin_p = jnp.zeros((H, H), jnp.float32).at[:, :OUTD].set(Wlin)
    blin_p = jnp.zeros((1, H), jnp.float32).at[0, :OUTD].set(blin)
    outp = _k3(agg2, deg2d, b2r, batch2d, wlin_p, blin_p)
    return outp[:, :OUTD]
